# baseline (speedup 1.0000x reference)
import functools

import jax
import jax.numpy as jnp
from jax import lax
from jax.experimental import pallas as pl
from jax.experimental.pallas import tpu as pltpu
from jax.experimental.pallas import tpu_sc as plsc

B = 4096
DENSE_DIM = 13
N_FIELDS = 26
VOCAB = 100000
EMBED_DIM = 32
BF = B * N_FIELDS
TAB_WORDS = N_FIELDS * VOCAB * EMBED_DIM

_LANES = 16
_CHUNK = 128
_Q_ITEMS = 832
_Q_WORDS = _Q_ITEMS * EMBED_DIM


def _sc_gather_make(num_workers: int, per_w: int):
  mesh = plsc.VectorSubcoreMesh(core_axis_name="c", subcore_axis_name="s")

  @functools.partial(
      pl.kernel,
      mesh=mesh,
      compiler_params=pltpu.CompilerParams(needs_layout_passes=False),
      out_type=jax.ShapeDtypeStruct((BF * EMBED_DIM,), jnp.float32),
      scratch_types=[
          pltpu.VMEM((per_w,), jnp.int32),
          pltpu.VMEM((_Q_WORDS,), jnp.int32),
          pltpu.VMEM((_Q_WORDS,), jnp.int32),
          pltpu.VMEM((_Q_WORDS,), jnp.float32),
          pltpu.VMEM((_Q_WORDS,), jnp.float32),
          pltpu.SemaphoreType.DMA,
          pltpu.SemaphoreType.DMA,
      ],
  )
  def gather_k(tab_hbm, sp_hbm, out_hbm, base_v, widx0, widx1, data0, data1,
               sem0, sem1):
    wid = lax.axis_index("s") * 2 + lax.axis_index("c")
    base = wid * per_w
    pltpu.sync_copy(sp_hbm.at[pl.ds(base, per_w)], base_v)
    iota = lax.iota(jnp.int32, _LANES)
    def base_body(k, carry):
      sl = pl.ds(k * _LANES, _LANES)
      pos = base + k * _LANES + iota
      base_v[sl] = base_v[sl] + lax.rem(pos, N_FIELDS) * (VOCAB * EMBED_DIM)
      return carry
    lax.fori_loop(0, per_w // _LANES, base_body, 0)

    bufs = [(widx0, data0, sem0), (widx1, data1, sem1)]
    n_chunks = _Q_WORDS // _CHUNK
    nq = per_w // _Q_ITEMS

    def expand(q, widx):
      qoff = q * _Q_ITEMS
      def body(k, carry):
        b16 = base_v[pl.ds(qoff + k * _LANES, _LANES)]
        pos = (k * _LANES + iota) * EMBED_DIM
        for e in range(EMBED_DIM):
          plsc.store_scatter(widx, [pos + e], b16 + e * VOCAB)
        return carry
      lax.fori_loop(0, _Q_ITEMS // _LANES, body, 0)

    def fire(widx, data, sem):
      def body(j, carry):
        sl = pl.ds(j * _CHUNK, _CHUNK)
        pltpu.async_copy(tab_hbm.at[widx.at[sl]], data.at[sl], sem)
        return carry
      lax.fori_loop(0, n_chunks, body, 0)

    def drain_write(q, widx, data, sem):
      def body(j, carry):
        sl = pl.ds(j * _CHUNK, _CHUNK)
        pltpu.make_async_copy(tab_hbm.at[widx.at[sl]], data.at[sl], sem).wait()
        return carry
      lax.fori_loop(0, n_chunks, body, 0)
      pltpu.sync_copy(
          data,
          out_hbm.at[pl.ds((base + q * _Q_ITEMS) * EMBED_DIM, _Q_WORDS)])

    for q in range(nq + 2):
      if q >= 2:
        w, d, s = bufs[(q - 2) % 2]
        drain_write(q - 2, w, d, s)
      if q < nq:
        w, d, s = bufs[q % 2]
        expand(q, w)
        fire(w, d, s)

  return gather_k


def _mlp_body(dense_ref, embs_ref, w1d_ref, w1e_ref, b1_ref, w2_ref, b2_ref,
              w3_ref, b3_ref, out_ref):
  x1 = (dense_ref[...] @ w1d_ref[...] + embs_ref[...] @ w1e_ref[...]
        + b1_ref[...])
  h1 = jnp.maximum(x1, 0.0)
  h2 = jnp.maximum(h1 @ w2_ref[...] + b2_ref[...], 0.0)
  o = h2 @ w3_ref[...] + b3_ref[...]
  out_ref[...] = jax.nn.sigmoid(o)


def kernel(dense, sparse, tables, W1, b1, W2, b2, W3, b3):
  tab_flat = jnp.transpose(tables, (0, 2, 1)).reshape(TAB_WORDS)
  sp_flat = sparse.reshape(BF)

  info = plsc.get_sparse_core_info()
  nw = info.num_cores * info.num_subcores
  per_w = BF // nw
  embs = _sc_gather_make(nw, per_w)(tab_flat, sp_flat)
  embs = embs.reshape(B, N_FIELDS * EMBED_DIM)

  w1d = W1[:DENSE_DIM]
  w1e = W1[DENSE_DIM:]
  bs = 512
  grid = (B // bs,)
  full = lambda shape: pl.BlockSpec(shape, lambda i: (0, 0))
  out = pl.pallas_call(
      _mlp_body,
      grid=grid,
      in_specs=[
          pl.BlockSpec((bs, DENSE_DIM), lambda i: (i, 0)),
          pl.BlockSpec((bs, N_FIELDS * EMBED_DIM), lambda i: (i, 0)),
          full(w1d.shape),
          full(w1e.shape),
          pl.BlockSpec((1, 128), lambda i: (0, 0)),
          full(W2.shape),
          pl.BlockSpec((1, 64), lambda i: (0, 0)),
          full(W3.shape),
          pl.BlockSpec((1, 1), lambda i: (0, 0)),
      ],
      out_specs=pl.BlockSpec((bs, 1), lambda i: (i, 0)),
      out_shape=jax.ShapeDtypeStruct((B, 1), jnp.float32),
  )(dense, embs, w1d, w1e, b1.reshape(1, 128), W2, b2.reshape(1, 64), W3,
    b3.reshape(1, 1))
  return out.reshape(B)


# speedup vs baseline: 1.0222x; 1.0222x over previous
import functools

import jax
import jax.numpy as jnp
from jax import lax
from jax.experimental import pallas as pl
from jax.experimental.pallas import tpu as pltpu
from jax.experimental.pallas import tpu_sc as plsc

B = 4096
DENSE_DIM = 13
N_FIELDS = 26
VOCAB = 100000
EMBED_DIM = 32
BF = B * N_FIELDS
TAB_WORDS = N_FIELDS * VOCAB * EMBED_DIM

_LANES = 16
_CHUNK = 128
_HALF_ITEMS = 1664
_HALF_WORDS = _HALF_ITEMS * EMBED_DIM


def _sc_gather_make(num_workers: int, per_w: int):
  mesh = plsc.VectorSubcoreMesh(core_axis_name="c", subcore_axis_name="s")

  @functools.partial(
      pl.kernel,
      mesh=mesh,
      compiler_params=pltpu.CompilerParams(needs_layout_passes=False),
      out_type=jax.ShapeDtypeStruct((BF * EMBED_DIM,), jnp.float32),
      scratch_types=[
          pltpu.VMEM((per_w,), jnp.int32),
          pltpu.VMEM((_HALF_WORDS,), jnp.int32),
          pltpu.VMEM((_HALF_WORDS,), jnp.float32),
          pltpu.SemaphoreType.DMA,
      ],
  )
  def gather_k(tab_hbm, sp_hbm, out_hbm, base_v, widx_v, data_v, sem):
    wid = lax.axis_index("s") * 2 + lax.axis_index("c")
    base = wid * per_w
    pltpu.sync_copy(sp_hbm.at[pl.ds(base, per_w)], base_v)
    iota = lax.iota(jnp.int32, _LANES)
    def base_body(k, carry):
      sl = pl.ds(k * _LANES, _LANES)
      pos = base + k * _LANES + iota
      base_v[sl] = base_v[sl] + lax.rem(pos, N_FIELDS) * (VOCAB * EMBED_DIM)
      return carry
    lax.fori_loop(0, per_w // _LANES, base_body, 0)

    c_lo = iota * VOCAB
    c_hi = (iota + _LANES) * VOCAB
    n_chunks = _HALF_WORDS // _CHUNK
    for h in range(per_w // _HALF_ITEMS):
      hoff = h * _HALF_ITEMS
      def expand(m, carry):
        bvec = plsc.load_gather(
            base_v, [jnp.broadcast_to(hoff + m, (_LANES,)).astype(jnp.int32)])
        widx_v[pl.ds(m * EMBED_DIM, _LANES)] = bvec + c_lo
        widx_v[pl.ds(m * EMBED_DIM + _LANES, _LANES)] = bvec + c_hi
        return carry
      lax.fori_loop(0, _HALF_ITEMS, expand, 0)
      def fire(j, carry):
        sl = pl.ds(j * _CHUNK, _CHUNK)
        pltpu.async_copy(tab_hbm.at[widx_v.at[sl]], data_v.at[sl], sem)
        return carry
      lax.fori_loop(0, n_chunks, fire, 0)
      def drain(j, carry):
        sl = pl.ds(j * _CHUNK, _CHUNK)
        pltpu.make_async_copy(
            tab_hbm.at[widx_v.at[sl]], data_v.at[sl], sem).wait()
        return carry
      lax.fori_loop(0, n_chunks, drain, 0)
      pltpu.sync_copy(
          data_v,
          out_hbm.at[pl.ds((base + hoff) * EMBED_DIM, _HALF_WORDS)])

  return gather_k


def _mlp_body(dense_ref, embs_ref, w1d_ref, w1e_ref, b1_ref, w2_ref, b2_ref,
              w3_ref, b3_ref, out_ref):
  x1 = (dense_ref[...] @ w1d_ref[...] + embs_ref[...] @ w1e_ref[...]
        + b1_ref[...])
  h1 = jnp.maximum(x1, 0.0)
  h2 = jnp.maximum(h1 @ w2_ref[...] + b2_ref[...], 0.0)
  o = h2 @ w3_ref[...] + b3_ref[...]
  out_ref[...] = jax.nn.sigmoid(o)


def kernel(dense, sparse, tables, W1, b1, W2, b2, W3, b3):
  tab_flat = jnp.transpose(tables, (0, 2, 1)).reshape(TAB_WORDS)
  sp_flat = sparse.reshape(BF)

  info = plsc.get_sparse_core_info()
  nw = info.num_cores * info.num_subcores
  per_w = BF // nw
  embs = _sc_gather_make(nw, per_w)(tab_flat, sp_flat)
  embs = embs.reshape(B, N_FIELDS * EMBED_DIM)

  w1d = W1[:DENSE_DIM]
  w1e = W1[DENSE_DIM:]
  bs = 512
  grid = (B // bs,)
  full = lambda shape: pl.BlockSpec(shape, lambda i: (0, 0))
  out = pl.pallas_call(
      _mlp_body,
      grid=grid,
      in_specs=[
          pl.BlockSpec((bs, DENSE_DIM), lambda i: (i, 0)),
          pl.BlockSpec((bs, N_FIELDS * EMBED_DIM), lambda i: (i, 0)),
          full(w1d.shape),
          full(w1e.shape),
          pl.BlockSpec((1, 128), lambda i: (0, 0)),
          full(W2.shape),
          pl.BlockSpec((1, 64), lambda i: (0, 0)),
          full(W3.shape),
          pl.BlockSpec((1, 1), lambda i: (0, 0)),
      ],
      out_specs=pl.BlockSpec((bs, 1), lambda i: (i, 0)),
      out_shape=jax.ShapeDtypeStruct((B, 1), jnp.float32),
  )(dense, embs, w1d, w1e, b1.reshape(1, 128), W2, b2.reshape(1, 64), W3,
    b3.reshape(1, 1))
  return out.reshape(B)


# speedup vs baseline: 1.0382x; 1.0157x over previous
import functools

import jax
import jax.numpy as jnp
from jax import lax
from jax.experimental import pallas as pl
from jax.experimental.pallas import tpu as pltpu
from jax.experimental.pallas import tpu_sc as plsc

B = 4096
DENSE_DIM = 13
N_FIELDS = 26
VOCAB = 100000
EMBED_DIM = 32
BF = B * N_FIELDS
TAB_WORDS = N_FIELDS * VOCAB * EMBED_DIM

_LANES = 16
_CHUNK = 128
_Q_ITEMS = 832
_Q_WORDS = _Q_ITEMS * EMBED_DIM


def _sc_gather_make(num_workers: int, per_w: int):
  mesh = plsc.VectorSubcoreMesh(core_axis_name="c", subcore_axis_name="s")

  @functools.partial(
      pl.kernel,
      mesh=mesh,
      compiler_params=pltpu.CompilerParams(needs_layout_passes=False),
      out_type=jax.ShapeDtypeStruct((BF * EMBED_DIM,), jnp.float32),
      scratch_types=[
          pltpu.VMEM((per_w,), jnp.int32),
          pltpu.VMEM((_Q_WORDS,), jnp.int32),
          pltpu.VMEM((_Q_WORDS,), jnp.int32),
          pltpu.VMEM((_Q_WORDS,), jnp.float32),
          pltpu.VMEM((_Q_WORDS,), jnp.float32),
          pltpu.SemaphoreType.DMA,
          pltpu.SemaphoreType.DMA,
      ],
  )
  def gather_k(tab_hbm, sp_hbm, out_hbm, base_v, widx0, widx1, data0, data1,
               sem0, sem1):
    wid = lax.axis_index("s") * 2 + lax.axis_index("c")
    base = wid * per_w
    pltpu.sync_copy(sp_hbm.at[pl.ds(base, per_w)], base_v)
    iota = lax.iota(jnp.int32, _LANES)
    def base_body(k, carry):
      sl = pl.ds(k * _LANES, _LANES)
      pos = base + k * _LANES + iota
      base_v[sl] = base_v[sl] + lax.rem(pos, N_FIELDS) * (VOCAB * EMBED_DIM)
      return carry
    lax.fori_loop(0, per_w // _LANES, base_body, 0)

    c_lo = iota * VOCAB
    c_hi = (iota + _LANES) * VOCAB
    n_chunks = _Q_WORDS // _CHUNK
    nq = per_w // _Q_ITEMS
    widxs = [widx0, widx1]
    datas = [data0, data1]
    sems = [sem0, sem1]

    def expand(q, widx):
      qoff = q * _Q_ITEMS
      def body(m, carry):
        bvec = plsc.load_gather(
            base_v, [jnp.broadcast_to(qoff + m, (_LANES,)).astype(jnp.int32)])
        widx[pl.ds(m * EMBED_DIM, _LANES)] = bvec + c_lo
        widx[pl.ds(m * EMBED_DIM + _LANES, _LANES)] = bvec + c_hi
        return carry
      lax.fori_loop(0, _Q_ITEMS, body, 0)

    def fire(widx, data, sem):
      def body(j, carry):
        sl = pl.ds(j * _CHUNK, _CHUNK)
        pltpu.async_copy(tab_hbm.at[widx.at[sl]], data.at[sl], sem)
        return carry
      lax.fori_loop(0, n_chunks, body, 0)

    def drain_write(q, widx, data, sem):
      def body(j, carry):
        sl = pl.ds(j * _CHUNK, _CHUNK)
        pltpu.make_async_copy(tab_hbm.at[widx.at[sl]], data.at[sl], sem).wait()
        return carry
      lax.fori_loop(0, n_chunks, body, 0)
      pltpu.sync_copy(
          data, out_hbm.at[pl.ds((base + q * _Q_ITEMS) * EMBED_DIM, _Q_WORDS)])

    expand(0, widxs[0])
    fire(widxs[0], datas[0], sems[0])
    expand(1, widxs[1])
    for q in range(nq):
      if q + 1 < nq:
        fire(widxs[(q + 1) % 2], datas[(q + 1) % 2], sems[(q + 1) % 2])
      drain_write(q, widxs[q % 2], datas[q % 2], sems[q % 2])
      if q + 2 < nq:
        expand(q + 2, widxs[q % 2])

  return gather_k


def _mlp_body(dense_ref, embs_ref, w1d_ref, w1e_ref, b1_ref, w2_ref, b2_ref,
              w3_ref, b3_ref, out_ref):
  x1 = (dense_ref[...] @ w1d_ref[...] + embs_ref[...] @ w1e_ref[...]
        + b1_ref[...])
  h1 = jnp.maximum(x1, 0.0)
  h2 = jnp.maximum(h1 @ w2_ref[...] + b2_ref[...], 0.0)
  o = h2 @ w3_ref[...] + b3_ref[...]
  out_ref[...] = jax.nn.sigmoid(o)


def kernel(dense, sparse, tables, W1, b1, W2, b2, W3, b3):
  tab_flat = jnp.transpose(tables, (0, 2, 1)).reshape(TAB_WORDS)
  sp_flat = sparse.reshape(BF)

  info = plsc.get_sparse_core_info()
  nw = info.num_cores * info.num_subcores
  per_w = BF // nw
  embs = _sc_gather_make(nw, per_w)(tab_flat, sp_flat)
  embs = embs.reshape(B, N_FIELDS * EMBED_DIM)

  w1d = W1[:DENSE_DIM]
  w1e = W1[DENSE_DIM:]
  bs = 512
  grid = (B // bs,)
  full = lambda shape: pl.BlockSpec(shape, lambda i: (0, 0))
  out = pl.pallas_call(
      _mlp_body,
      grid=grid,
      in_specs=[
          pl.BlockSpec((bs, DENSE_DIM), lambda i: (i, 0)),
          pl.BlockSpec((bs, N_FIELDS * EMBED_DIM), lambda i: (i, 0)),
          full(w1d.shape),
          full(w1e.shape),
          pl.BlockSpec((1, 128), lambda i: (0, 0)),
          full(W2.shape),
          pl.BlockSpec((1, 64), lambda i: (0, 0)),
          full(W3.shape),
          pl.BlockSpec((1, 1), lambda i: (0, 0)),
      ],
      out_specs=pl.BlockSpec((bs, 1), lambda i: (i, 0)),
      out_shape=jax.ShapeDtypeStruct((B, 1), jnp.float32),
  )(dense, embs, w1d, w1e, b1.reshape(1, 128), W2, b2.reshape(1, 64), W3,
    b3.reshape(1, 1))
  return out.reshape(B)
